# Initial kernel scaffold; baseline (speedup 1.0000x reference)
import jax
import jax.numpy as jnp
from jax.experimental import pallas as pl


def kernel(queries, keys):
    raise NotImplementedError("write your pallas kernel here")


# speedup vs baseline: 1.4588x; 1.4588x over previous
import functools

import jax
import jax.numpy as jnp
from jax.experimental import pallas as pl
from jax.experimental.pallas import tpu as pltpu

N_NEIGHBOURS = 5
TQ = 256
BK = 1024
K_REAL = 100000
K_PAD = 100352
PAD_VAL = 1e4
BIGF = 3e38
BIGI = 2**30


def _knn_kernel(q_ref, k_ref, scores_ref, idx_ref, rv_ref, ri_ref):
    ki = pl.program_id(1)
    nk = pl.num_programs(1)

    @pl.when(ki == 0)
    def _init():
        rv_ref[...] = jnp.full((TQ, 128), BIGF, jnp.float32)
        ri_ref[...] = jnp.full((TQ, 128), BIGI, jnp.int32)

    q = q_ref[...]
    k = k_ref[...]
    q2 = jnp.sum(q * q, axis=1, keepdims=True)
    k2 = jnp.sum(k * k, axis=1)
    dots = jax.lax.dot_general(
        q, k, (((1,), (1,)), ((), ())),
        preferred_element_type=jnp.float32)
    dist = q2 + k2[None, :] - 2.0 * dots

    gidx = ki * BK + jax.lax.broadcasted_iota(jnp.int32, (TQ, BK), 1)

    cv = jnp.concatenate([rv_ref[...], dist], axis=1)
    cidx = jnp.concatenate([ri_ref[...], gidx], axis=1)
    pos_iota = jax.lax.broadcasted_iota(jnp.int32, cv.shape, 1)

    new_v = []
    new_i = []
    for _ in range(N_NEIGHBOURS):
        m = jnp.min(cv, axis=1, keepdims=True)
        pos = jnp.min(jnp.where(cv == m, pos_iota, BIGI), axis=1,
                      keepdims=True)
        sel = pos_iota == pos
        new_v.append(m)
        new_i.append(jnp.max(jnp.where(sel, cidx, -1), axis=1, keepdims=True))
        cv = jnp.where(sel, BIGF, cv)

    vals5 = jnp.concatenate(new_v, axis=1)
    idx5 = jnp.concatenate(new_i, axis=1)

    pad_v = jnp.full((TQ, 128 - N_NEIGHBOURS), BIGF, jnp.float32)
    pad_i = jnp.full((TQ, 128 - N_NEIGHBOURS), BIGI, jnp.int32)
    rv_ref[...] = jnp.concatenate([vals5, pad_v], axis=1)
    ri_ref[...] = jnp.concatenate([idx5, pad_i], axis=1)

    @pl.when(ki == nk - 1)
    def _emit():
        scores_ref[...] = jnp.mean(vals5, axis=1, keepdims=True)
        idx_ref[...] = idx5


@jax.jit
def kernel(queries, keys):
    Q, D = queries.shape
    K, _ = keys.shape
    keys_p = jnp.pad(keys, ((0, K_PAD - K), (0, 0)), constant_values=PAD_VAL)

    grid = (Q // TQ, K_PAD // BK)
    scores2d, topk_idx = pl.pallas_call(
        _knn_kernel,
        grid=grid,
        in_specs=[
            pl.BlockSpec((TQ, D), lambda qi, ki: (qi, 0)),
            pl.BlockSpec((BK, D), lambda qi, ki: (ki, 0)),
        ],
        out_specs=[
            pl.BlockSpec((TQ, 1), lambda qi, ki: (qi, 0)),
            pl.BlockSpec((TQ, N_NEIGHBOURS), lambda qi, ki: (qi, 0)),
        ],
        out_shape=[
            jax.ShapeDtypeStruct((Q, 1), jnp.float32),
            jax.ShapeDtypeStruct((Q, N_NEIGHBOURS), jnp.int32),
        ],
        scratch_shapes=[
            pltpu.VMEM((TQ, 128), jnp.float32),
            pltpu.VMEM((TQ, 128), jnp.int32),
        ],
        compiler_params=pltpu.CompilerParams(
            dimension_semantics=("parallel", "arbitrary"),
        ),
    )(queries, keys_p)
    return scores2d[:, 0], topk_idx


# speedup vs baseline: 2.1764x; 1.4919x over previous
import functools

import jax
import jax.numpy as jnp
from jax.experimental import pallas as pl
from jax.experimental.pallas import tpu as pltpu

N_NEIGHBOURS = 5
TQ = 256
BK = 1024
K_REAL = 100000
K_PAD = 100352
PAD_VAL = 1e4
BIGF = 3e38
BIGI = 2**30


def _knn_kernel(q_ref, k_ref, scores_ref, idx_ref, rv_ref, ri_ref):
    ki = pl.program_id(1)
    nk = pl.num_programs(1)

    @pl.when(ki == 0)
    def _init():
        rv_ref[...] = jnp.full((TQ, 128), BIGF, jnp.float32)
        ri_ref[...] = jnp.full((TQ, 128), BIGI, jnp.int32)

    q = q_ref[...]
    k = k_ref[...]
    q2 = jnp.sum(q * q, axis=1, keepdims=True)
    k2 = jnp.sum(k * k, axis=1)
    dots = jax.lax.dot_general(
        q, k, (((1,), (1,)), ((), ())),
        preferred_element_type=jnp.float32)
    dist = q2 + k2[None, :] - 2.0 * dots

    cv = jnp.concatenate([rv_ref[...], dist], axis=1)
    posf = jax.lax.broadcasted_iota(jnp.int32, cv.shape, 1).astype(jnp.float32)
    run_idx = ri_ref[...]
    lane_f = jax.lax.broadcasted_iota(
        jnp.int32, (TQ, 128), 1).astype(jnp.float32)

    new_v = []
    new_i = []
    for _ in range(N_NEIGHBOURS):
        m = jnp.min(cv, axis=1, keepdims=True)
        key = jnp.where(cv == m, posf, BIGF)
        pos = jnp.min(key, axis=1, keepdims=True)
        cv = jnp.where(key == pos, BIGF, cv)
        rg = jnp.min(jnp.where(lane_f == pos, run_idx, BIGI), axis=1,
                     keepdims=True)
        blk = (ki * BK - 128) + pos.astype(jnp.int32)
        new_v.append(m)
        new_i.append(jnp.where(pos < 128.0, rg, blk))

    vals5 = jnp.concatenate(new_v, axis=1)
    idx5 = jnp.concatenate(new_i, axis=1)

    pad_v = jnp.full((TQ, 128 - N_NEIGHBOURS), BIGF, jnp.float32)
    pad_i = jnp.full((TQ, 128 - N_NEIGHBOURS), BIGI, jnp.int32)
    rv_ref[...] = jnp.concatenate([vals5, pad_v], axis=1)
    ri_ref[...] = jnp.concatenate([idx5, pad_i], axis=1)

    @pl.when(ki == nk - 1)
    def _emit():
        scores_ref[...] = jnp.mean(vals5, axis=1, keepdims=True)
        idx_ref[...] = idx5


@jax.jit
def kernel(queries, keys):
    Q, D = queries.shape
    K, _ = keys.shape
    keys_p = jnp.pad(keys, ((0, K_PAD - K), (0, 0)), constant_values=PAD_VAL)

    grid = (Q // TQ, K_PAD // BK)
    scores2d, topk_idx = pl.pallas_call(
        _knn_kernel,
        grid=grid,
        in_specs=[
            pl.BlockSpec((TQ, D), lambda qi, ki: (qi, 0)),
            pl.BlockSpec((BK, D), lambda qi, ki: (ki, 0)),
        ],
        out_specs=[
            pl.BlockSpec((TQ, 1), lambda qi, ki: (qi, 0)),
            pl.BlockSpec((TQ, N_NEIGHBOURS), lambda qi, ki: (qi, 0)),
        ],
        out_shape=[
            jax.ShapeDtypeStruct((Q, 1), jnp.float32),
            jax.ShapeDtypeStruct((Q, N_NEIGHBOURS), jnp.int32),
        ],
        scratch_shapes=[
            pltpu.VMEM((TQ, 128), jnp.float32),
            pltpu.VMEM((TQ, 128), jnp.int32),
        ],
        compiler_params=pltpu.CompilerParams(
            dimension_semantics=("parallel", "arbitrary"),
        ),
    )(queries, keys_p)
    return scores2d[:, 0], topk_idx


# speedup vs baseline: 2.3337x; 1.0723x over previous
import functools

import jax
import jax.numpy as jnp
from jax import lax
from jax.experimental import pallas as pl
from jax.experimental.pallas import tpu as pltpu
from jax.experimental.pallas import tpu_sc as plsc

N_NEIGHBOURS = 5
TQ = 256
BK = 1024
NB = 98
K_PAD = 100352
PAD_VAL = 1e4
BIGF = 3e38

SC_NC = 2
SC_NS = 16
SC_NW = SC_NC * SC_NS
SC_CHUNK = 64


def _dist_kernel(q_ref, k_ref, dist_ref, bm_ref, acc_ref):
    ki = pl.program_id(1)

    @pl.when(ki == 0)
    def _init():
        acc_ref[...] = jnp.full((TQ, 128), BIGF, jnp.float32)

    q = q_ref[...]
    k = k_ref[...]
    q2 = jnp.sum(q * q, axis=1, keepdims=True)
    k2 = jnp.sum(k * k, axis=1)
    dots = jax.lax.dot_general(
        q, k, (((1,), (1,)), ((), ())),
        preferred_element_type=jnp.float32)
    dist = q2 + k2[None, :] - 2.0 * dots
    dist_ref[...] = dist
    m = jnp.min(dist, axis=1, keepdims=True)
    lane = jax.lax.broadcasted_iota(jnp.int32, (TQ, 128), 1)
    acc_ref[...] = jnp.where(lane == ki, m, acc_ref[...])

    @pl.when(ki == NB - 1)
    def _emit():
        bm_ref[...] = acc_ref[...]


def _select_kernel(bm_ref, cb_ref, fi_ref):
    qi = pl.program_id(0)
    lane = jax.lax.broadcasted_iota(jnp.int32, (TQ, 128), 1)
    lane_f = lane.astype(jnp.float32)
    cv = jnp.where(lane < NB, bm_ref[...], BIGF)
    cols = []
    for _ in range(N_NEIGHBOURS):
        m = jnp.min(cv, axis=1, keepdims=True)
        key = jnp.where(cv == m, lane_f, BIGF)
        pos = jnp.min(key, axis=1, keepdims=True)
        cv = jnp.where(key == pos, BIGF, cv)
        cols.append(pos)
    for a, b in ((0, 1), (3, 4), (2, 4), (2, 3), (1, 4),
                 (0, 3), (0, 2), (1, 3), (1, 2)):
        lo = jnp.minimum(cols[a], cols[b])
        hi = jnp.maximum(cols[a], cols[b])
        cols[a], cols[b] = lo, hi
    cb = jnp.concatenate(cols, axis=1).astype(jnp.int32)
    cb_ref[...] = cb
    row = qi * TQ + jax.lax.broadcasted_iota(jnp.int32, (TQ, 1), 0)
    fi_ref[...] = row * NB + cb


def _sc_gather(table_ref, idx_ref, out_ref, idx_v, rows_v, sem):
    b_per_w = (4096 * N_NEIGHBOURS) // SC_NW
    wid = lax.axis_index("s") * SC_NC + lax.axis_index("c")
    base = wid * b_per_w
    for c in range(b_per_w // SC_CHUNK):
        off = base + c * SC_CHUNK
        pltpu.sync_copy(idx_ref.at[pl.ds(off, SC_CHUNK)], idx_v)
        pltpu.async_copy(table_ref.at[idx_v], rows_v, sem).wait()
        pltpu.sync_copy(rows_v, out_ref.at[pl.ds(off, SC_CHUNK)])


def _final_kernel(g_ref, cb_ref, scores_ref, idx_ref):
    g = g_ref[...]
    cbf = cb_ref[...].astype(jnp.float32)
    posf = jax.lax.broadcasted_iota(jnp.int32, g.shape, 1).astype(jnp.float32)
    col5 = jax.lax.broadcasted_iota(
        jnp.int32, (TQ, N_NEIGHBOURS), 1).astype(jnp.float32)
    vals = []
    idxs = []
    for _ in range(N_NEIGHBOURS):
        m = jnp.min(g, axis=1, keepdims=True)
        key = jnp.where(g == m, posf, BIGF)
        pos = jnp.min(key, axis=1, keepdims=True)
        g = jnp.where(key == pos, BIGF, g)
        j = jnp.floor(pos * (1.0 / BK))
        lanep = pos - j * BK
        cbj = jnp.min(jnp.where(col5 == j, cbf, BIGF), axis=1, keepdims=True)
        vals.append(m)
        idxs.append(cbj * BK + lanep)
    vals5 = jnp.concatenate(vals, axis=1)
    scores_ref[...] = jnp.mean(vals5, axis=1, keepdims=True)
    idx_ref[...] = jnp.concatenate(idxs, axis=1).astype(jnp.int32)


def _gather_candidates(dist_flat, fi_flat):
    nrows = fi_flat.shape[0]
    gather = pl.kernel(
        _sc_gather,
        out_type=jax.ShapeDtypeStruct((nrows, BK), jnp.float32),
        mesh=plsc.VectorSubcoreMesh(core_axis_name="c", subcore_axis_name="s"),
        scratch_types=[
            pltpu.VMEM((SC_CHUNK,), jnp.int32),
            pltpu.VMEM((SC_CHUNK, BK), jnp.float32),
            pltpu.SemaphoreType.DMA,
        ],
    )
    return gather(dist_flat, fi_flat)


@jax.jit
def kernel(queries, keys):
    Q, D = queries.shape
    K, _ = keys.shape
    keys_p = jnp.pad(keys, ((0, K_PAD - K), (0, 0)), constant_values=PAD_VAL)

    dist, bm = pl.pallas_call(
        _dist_kernel,
        grid=(Q // TQ, NB),
        in_specs=[
            pl.BlockSpec((TQ, D), lambda qi, ki: (qi, 0)),
            pl.BlockSpec((BK, D), lambda qi, ki: (ki, 0)),
        ],
        out_specs=[
            pl.BlockSpec((TQ, BK), lambda qi, ki: (qi, ki)),
            pl.BlockSpec((TQ, 128), lambda qi, ki: (qi, 0)),
        ],
        out_shape=[
            jax.ShapeDtypeStruct((Q, K_PAD), jnp.float32),
            jax.ShapeDtypeStruct((Q, 128), jnp.float32),
        ],
        scratch_shapes=[pltpu.VMEM((TQ, 128), jnp.float32)],
        compiler_params=pltpu.CompilerParams(
            dimension_semantics=("parallel", "arbitrary"),
        ),
    )(queries, keys_p)

    cb, fi = pl.pallas_call(
        _select_kernel,
        grid=(Q // TQ,),
        in_specs=[pl.BlockSpec((TQ, 128), lambda qi: (qi, 0))],
        out_specs=[
            pl.BlockSpec((TQ, N_NEIGHBOURS), lambda qi: (qi, 0)),
            pl.BlockSpec((TQ, N_NEIGHBOURS), lambda qi: (qi, 0)),
        ],
        out_shape=[
            jax.ShapeDtypeStruct((Q, N_NEIGHBOURS), jnp.int32),
            jax.ShapeDtypeStruct((Q, N_NEIGHBOURS), jnp.int32),
        ],
    )(bm)

    gathered = _gather_candidates(
        dist.reshape(Q * NB, BK), fi.reshape(Q * N_NEIGHBOURS))

    scores2d, topk_idx = pl.pallas_call(
        _final_kernel,
        grid=(Q // TQ,),
        in_specs=[
            pl.BlockSpec((TQ, N_NEIGHBOURS * BK), lambda qi: (qi, 0)),
            pl.BlockSpec((TQ, N_NEIGHBOURS), lambda qi: (qi, 0)),
        ],
        out_specs=[
            pl.BlockSpec((TQ, 1), lambda qi: (qi, 0)),
            pl.BlockSpec((TQ, N_NEIGHBOURS), lambda qi: (qi, 0)),
        ],
        out_shape=[
            jax.ShapeDtypeStruct((Q, 1), jnp.float32),
            jax.ShapeDtypeStruct((Q, N_NEIGHBOURS), jnp.int32),
        ],
    )(gathered.reshape(Q, N_NEIGHBOURS * BK), cb)
    return scores2d[:, 0], topk_idx


# speedup vs baseline: 3.8129x; 1.6339x over previous
import functools

import jax
import jax.numpy as jnp
from jax import lax
from jax.experimental import pallas as pl
from jax.experimental.pallas import tpu as pltpu
from jax.experimental.pallas import tpu_sc as plsc

N_NEIGHBOURS = 5
TQ = 256
BK = 1024
NB = 98
K_PAD = 100352
NQ = 4096
PAD_VAL = 1e4
BIGF = 3e38

SC_NC = 2
SC_NS = 16
SC_NW = SC_NC * SC_NS
SC_CHUNK = 64


def _dist_kernel(q_ref, k_ref, dist_ref, bm_ref, acc_ref):
    ki = pl.program_id(1)

    @pl.when(ki == 0)
    def _init():
        acc_ref[...] = jnp.full((TQ, 128), BIGF, jnp.float32)

    q = q_ref[...]
    k = k_ref[...]
    q2 = jnp.sum(q * q, axis=1, keepdims=True)
    k2 = jnp.sum(k * k, axis=1)
    dots = jax.lax.dot_general(
        q, k, (((1,), (1,)), ((), ())),
        preferred_element_type=jnp.float32)
    dist = q2 + k2[None, :] - 2.0 * dots
    dist_ref[...] = dist[None]
    m = jnp.min(dist, axis=1, keepdims=True)
    lane = jax.lax.broadcasted_iota(jnp.int32, (TQ, 128), 1)
    acc_ref[...] = jnp.where(lane == ki, m, acc_ref[...])

    @pl.when(ki == NB - 1)
    def _emit():
        bm_ref[...] = acc_ref[...]


def _select_kernel(bm_ref, cb_ref, fi_ref):
    qi = pl.program_id(0)
    lane = jax.lax.broadcasted_iota(jnp.int32, (TQ, 128), 1)
    lane_f = lane.astype(jnp.float32)
    cv = jnp.where(lane < NB, bm_ref[...], BIGF)
    cols = []
    for _ in range(N_NEIGHBOURS):
        m = jnp.min(cv, axis=1, keepdims=True)
        key = jnp.where(cv == m, lane_f, BIGF)
        pos = jnp.min(key, axis=1, keepdims=True)
        cv = jnp.where(key == pos, BIGF, cv)
        cols.append(pos)
    for a, b in ((0, 1), (3, 4), (2, 4), (2, 3), (1, 4),
                 (0, 3), (0, 2), (1, 3), (1, 2)):
        lo = jnp.minimum(cols[a], cols[b])
        hi = jnp.maximum(cols[a], cols[b])
        cols[a], cols[b] = lo, hi
    cb = jnp.concatenate(cols, axis=1).astype(jnp.int32)
    cb_ref[...] = cb
    row = qi * TQ + jax.lax.broadcasted_iota(jnp.int32, (TQ, 1), 0)
    fi_ref[...] = cb * NQ + row


def _sc_gather(table_ref, idx_ref, out_ref, idx_v, rows_v, sem):
    b_per_w = (NQ * N_NEIGHBOURS) // SC_NW
    wid = lax.axis_index("s") * SC_NC + lax.axis_index("c")
    base = wid * b_per_w
    for c in range(b_per_w // SC_CHUNK):
        off = base + c * SC_CHUNK
        pltpu.sync_copy(idx_ref.at[pl.ds(off, SC_CHUNK)], idx_v)
        pltpu.async_copy(table_ref.at[idx_v], rows_v, sem).wait()
        pltpu.sync_copy(rows_v, out_ref.at[pl.ds(off, SC_CHUNK)])


def _final_kernel(g_ref, cb_ref, scores_ref, idx_ref):
    g = [g_ref[j] for j in range(N_NEIGHBOURS)]
    cbf = cb_ref[...].astype(jnp.float32)
    lane_f = jax.lax.broadcasted_iota(
        jnp.int32, (TQ, BK), 1).astype(jnp.float32)
    gpos = [cbf[:, j:j + 1] * BK + lane_f for j in range(N_NEIGHBOURS)]
    vals = []
    idxs = []
    for _ in range(N_NEIGHBOURS):
        s = g[0]
        for j in range(1, N_NEIGHBOURS):
            s = jnp.minimum(s, g[j])
        m = jnp.min(s, axis=1, keepdims=True)
        keys = [jnp.where(g[j] == m, gpos[j], BIGF) for j in range(N_NEIGHBOURS)]
        ks = keys[0]
        for j in range(1, N_NEIGHBOURS):
            ks = jnp.minimum(ks, keys[j])
        pos = jnp.min(ks, axis=1, keepdims=True)
        g = [jnp.where(keys[j] == pos, BIGF, g[j]) for j in range(N_NEIGHBOURS)]
        vals.append(m)
        idxs.append(pos)
    vals5 = jnp.concatenate(vals, axis=1)
    scores_ref[...] = jnp.mean(vals5, axis=1, keepdims=True)
    idx_ref[...] = jnp.concatenate(idxs, axis=1).astype(jnp.int32)


def _gather_candidates(dist_flat, fi_flat):
    nrows = fi_flat.shape[0]
    gather = pl.kernel(
        _sc_gather,
        out_type=jax.ShapeDtypeStruct((nrows, BK), jnp.float32),
        mesh=plsc.VectorSubcoreMesh(core_axis_name="c", subcore_axis_name="s"),
        scratch_types=[
            pltpu.VMEM((SC_CHUNK,), jnp.int32),
            pltpu.VMEM((SC_CHUNK, BK), jnp.float32),
            pltpu.SemaphoreType.DMA,
        ],
    )
    return gather(dist_flat, fi_flat)


@jax.jit
def kernel(queries, keys):
    Q, D = queries.shape
    K, _ = keys.shape
    keys_p = jnp.pad(keys, ((0, K_PAD - K), (0, 0)), constant_values=PAD_VAL)

    dist, bm = pl.pallas_call(
        _dist_kernel,
        grid=(Q // TQ, NB),
        in_specs=[
            pl.BlockSpec((TQ, D), lambda qi, ki: (qi, 0)),
            pl.BlockSpec((BK, D), lambda qi, ki: (ki, 0)),
        ],
        out_specs=[
            pl.BlockSpec((1, TQ, BK), lambda qi, ki: (ki, qi, 0)),
            pl.BlockSpec((TQ, 128), lambda qi, ki: (qi, 0)),
        ],
        out_shape=[
            jax.ShapeDtypeStruct((NB, Q, BK), jnp.float32),
            jax.ShapeDtypeStruct((Q, 128), jnp.float32),
        ],
        scratch_shapes=[pltpu.VMEM((TQ, 128), jnp.float32)],
        compiler_params=pltpu.CompilerParams(
            dimension_semantics=("parallel", "arbitrary"),
        ),
    )(queries, keys_p)

    cb, fi = pl.pallas_call(
        _select_kernel,
        grid=(Q // TQ,),
        in_specs=[pl.BlockSpec((TQ, 128), lambda qi: (qi, 0))],
        out_specs=[
            pl.BlockSpec((TQ, N_NEIGHBOURS), lambda qi: (qi, 0)),
            pl.BlockSpec((TQ, N_NEIGHBOURS), lambda qi: (qi, 0)),
        ],
        out_shape=[
            jax.ShapeDtypeStruct((Q, N_NEIGHBOURS), jnp.int32),
            jax.ShapeDtypeStruct((Q, N_NEIGHBOURS), jnp.int32),
        ],
    )(bm)

    gathered = _gather_candidates(
        dist.reshape(NB * Q, BK), fi.T.reshape(Q * N_NEIGHBOURS))

    scores2d, topk_idx = pl.pallas_call(
        _final_kernel,
        grid=(Q // TQ,),
        in_specs=[
            pl.BlockSpec((N_NEIGHBOURS, TQ, BK), lambda qi: (0, qi, 0)),
            pl.BlockSpec((TQ, N_NEIGHBOURS), lambda qi: (qi, 0)),
        ],
        out_specs=[
            pl.BlockSpec((TQ, 1), lambda qi: (qi, 0)),
            pl.BlockSpec((TQ, N_NEIGHBOURS), lambda qi: (qi, 0)),
        ],
        out_shape=[
            jax.ShapeDtypeStruct((Q, 1), jnp.float32),
            jax.ShapeDtypeStruct((Q, N_NEIGHBOURS), jnp.int32),
        ],
    )(gathered.reshape(N_NEIGHBOURS, Q, BK), cb)
    return scores2d[:, 0], topk_idx
